# baseline (speedup 1.0000x reference)
import functools

import jax
import jax.numpy as jnp
from jax import lax
from jax.experimental import pallas as pl
from jax.experimental.pallas import tpu as pltpu
from jax.experimental.pallas import tpu_sc as plsc

NC = 2
NS = 16
NW = NC * NS
CK = 32
NRB = 8
NB = 12
LANES = 16


ZROWS = CK


def _make_sc_aggregate(n_pad: int, n_chunks: int, feat: int):
  rows_per_tile = n_pad // NS
  mesh = plsc.VectorSubcoreMesh(
      core_axis_name="c", subcore_axis_name="s", num_cores=NC,
      num_subcores=NS)

  @functools.partial(
      pl.kernel,
      out_type=jax.ShapeDtypeStruct((NC * n_pad, feat), jnp.float32),
      mesh=mesh,
      scratch_types=[
          pltpu.VMEM((NB, 2, CK), jnp.int32),
          pltpu.VMEM((NB, CK), jnp.float32),
          pltpu.VMEM((NRB, CK, feat), jnp.float32),
          pltpu.VMEM_SHARED((n_pad, feat), jnp.float32),
          pltpu.SemaphoreType.DMA,
          pltpu.SemaphoreType.DMA,
          pltpu.SemaphoreType.DMA,
      ],
  )
  def sc_aggregate(h_hbm, eidx_hbm, ew_hbm, out_hbm, ebuf, wbuf, rows_v,
                   agg_sh, esem, gsem, ssem):
    cid = lax.axis_index("c")
    sid = lax.axis_index("s")
    wid = sid * NC + cid

    def fetch_idx(c):
      pltpu.async_copy(eidx_hbm.at[wid, c], ebuf.at[lax.rem(c, NB)], esem)
      pltpu.async_copy(ew_hbm.at[wid, c], wbuf.at[lax.rem(c, NB)], esem)

    def wait_idx(c):
      pltpu.make_async_copy(eidx_hbm.at[wid, c],
                            ebuf.at[lax.rem(c, NB)], esem).wait()
      pltpu.make_async_copy(ew_hbm.at[wid, c],
                            wbuf.at[lax.rem(c, NB)], esem).wait()

    def start_gather(c, b):
      pltpu.async_copy(h_hbm.at[ebuf.at[lax.rem(c, NB), 0]],
                       rows_v.at[b], gsem)

    def wait_gather(c, b):
      pltpu.make_async_copy(h_hbm.at[ebuf.at[lax.rem(c, NB), 0]],
                            rows_v.at[b], gsem).wait()

    def start_scatter(c, b):
      pltpu.async_copy(rows_v.at[b], agg_sh.at[ebuf.at[lax.rem(c, NB), 1]],
                       ssem, add=True)

    def wait_scatter(c, b):
      pltpu.make_async_copy(rows_v.at[b],
                            agg_sh.at[ebuf.at[lax.rem(c, NB), 1]],
                            ssem).wait()

    zvec = jnp.zeros((LANES,), jnp.float32)

    def zfill(i, carry):
      for k in range(feat // LANES):
        rows_v[0, i, pl.ds(k * LANES, LANES)] = zvec
      return carry

    lax.fori_loop(0, ZROWS, zfill, 0)
    for i in range(rows_per_tile // ZROWS):
      pltpu.sync_copy(rows_v.at[0], agg_sh.at[pl.ds(sid * rows_per_tile
                                                    + i * ZROWS, ZROWS)])
    plsc.subcore_barrier()

    for c in range(min(6, n_chunks)):
      fetch_idx(c)
    for c in range(min(4, n_chunks)):
      wait_idx(c)
      start_gather(c, c)

    def chunk_body(c, carry):
      b = lax.rem(c, NRB)
      wait_gather(c, b)

      @pl.when(c >= 3)
      def _():
        wait_scatter(c - 3, lax.rem(c - 3, NRB))

      @pl.when(c + 4 < n_chunks)
      def _():
        wait_idx(c + 4)
        start_gather(c + 4, lax.rem(c + 4, NRB))

      @pl.when(c + 6 < n_chunks)
      def _():
        fetch_idx(c + 6)

      wslot = lax.rem(c, NB)

      @plsc.parallel_loop(0, CK)
      def scale_edge(e):
        g = lax.div(e, LANES) * LANES
        w16 = wbuf[wslot, pl.ds(g, LANES)]
        wj = lax.gather(
            w16, jnp.full((LANES, 1), e - g, jnp.int32),
            lax.GatherDimensionNumbers(offset_dims=(),
                                       collapsed_slice_dims=(0,),
                                       start_index_map=(0,)),
            slice_sizes=(1,),
            mode=lax.GatherScatterMode.PROMISE_IN_BOUNDS)
        vals = [rows_v[b, e, pl.ds(k * LANES, LANES)]
                for k in range(feat // LANES)]
        prods = [v * wj for v in vals]
        for k in range(feat // LANES):
          rows_v[b, e, pl.ds(k * LANES, LANES)] = prods[k]

      start_scatter(c, b)
      return carry

    lax.fori_loop(0, n_chunks, chunk_body, 0)
    for c in range(max(0, n_chunks - 3), n_chunks):
      wait_scatter(c, c % NRB)
    plsc.subcore_barrier()

    pltpu.sync_copy(
        agg_sh.at[pl.ds(sid * rows_per_tile, rows_per_tile)],
        out_hbm.at[pl.ds(cid * n_pad + sid * rows_per_tile,
                         rows_per_tile)])

  return sc_aggregate


_BLK = 512


def _layer_body(h_ref, a0_ref, a1_ref, w_ref, b_ref, o_ref):
  x = h_ref[...] + a0_ref[...] + a1_ref[...]
  y = lax.dot_general(x, w_ref[...], (((1,), (1,)), ((), ())),
                      preferred_element_type=jnp.float32)
  o_ref[...] = jnp.maximum(y + b_ref[...], 0.0)


def _tc_layer(h, agg, w, b):
  n_pad, feat = h.shape
  grid = n_pad // _BLK
  off = n_pad // _BLK
  return pl.pallas_call(
      _layer_body,
      grid=(grid,),
      in_specs=[
          pl.BlockSpec((_BLK, feat), lambda i: (i, 0)),
          pl.BlockSpec((_BLK, feat), lambda i: (i, 0)),
          pl.BlockSpec((_BLK, feat), lambda i: (i + off, 0)),
          pl.BlockSpec(w.shape, lambda i: (0, 0)),
          pl.BlockSpec((1, feat), lambda i: (0, 0)),
      ],
      out_specs=pl.BlockSpec((_BLK, feat), lambda i: (i, 0)),
      out_shape=jax.ShapeDtypeStruct((n_pad, feat), jnp.float32),
  )(h, agg, agg, w, b.reshape(1, feat))


def _readout_body(h1_ref, h2_ref, h3_ref, wr_ref, br_ref, o_ref):
  feat = h1_ref.shape[1]
  dn = (((1,), (1,)), ((), ()))
  y = lax.dot_general(jnp.maximum(h1_ref[...], 0.0), wr_ref[:, 0:feat],
                      dn, preferred_element_type=jnp.float32)
  y += lax.dot_general(jnp.maximum(h2_ref[...], 0.0),
                       wr_ref[:, feat:2 * feat], dn,
                       preferred_element_type=jnp.float32)
  y += lax.dot_general(jnp.maximum(h3_ref[...], 0.0),
                       wr_ref[:, 2 * feat:3 * feat], dn,
                       preferred_element_type=jnp.float32)
  o_ref[...] = y + br_ref[...]


def _tc_readout(h1, h2, h3, wr, br):
  n, feat = h1.shape
  grid = n // _BLK
  return pl.pallas_call(
      _readout_body,
      grid=(grid,),
      in_specs=[
          pl.BlockSpec((_BLK, feat), lambda i: (i, 0)),
          pl.BlockSpec((_BLK, feat), lambda i: (i, 0)),
          pl.BlockSpec((_BLK, feat), lambda i: (i, 0)),
          pl.BlockSpec(wr.shape, lambda i: (0, 0)),
          pl.BlockSpec((1, feat), lambda i: (0, 0)),
      ],
      out_specs=pl.BlockSpec((_BLK, feat), lambda i: (i, 0)),
      out_shape=jax.ShapeDtypeStruct((n, feat), jnp.float32),
  )(h1, h2, h3, wr, br.reshape(1, feat))


def kernel(node_embed, edge_index, edge_weight, W0, b0, W1, b1, W2, b2,
           Wr, br):
  n, feat = node_embed.shape
  e = edge_index.shape[1]
  n_chunks = -(-e // (NW * CK))
  e_pad = NW * n_chunks * CK
  n_pad = -(-n // (NS * ZROWS)) * NS * ZROWS

  src = jnp.pad(edge_index[0], (0, e_pad - e)).reshape(NW, n_chunks, CK)
  dst = jnp.pad(edge_index[1], (0, e_pad - e)).reshape(NW, n_chunks, CK)
  eidx = jnp.stack([src, dst], axis=2)
  ew = jnp.pad(edge_weight, (0, e_pad - e)).reshape(NW, n_chunks, CK)

  sc_aggregate = _make_sc_aggregate(n_pad, n_chunks, feat)

  def gin_layer(h, wmat, bvec):
    agg = sc_aggregate(h, eidx, ew)
    return _tc_layer(h, agg, wmat, bvec)

  h0 = jnp.pad(node_embed, ((0, n_pad - n), (0, 0)))
  h1 = gin_layer(h0, W0, b0)
  h2 = gin_layer(h1, W1, b1)
  h3 = gin_layer(h2, W2, b2)
  return _tc_readout(h1, h2, h3, Wr, br)[:n]


# speedup vs baseline: 1.0004x; 1.0004x over previous
import functools

import jax
import jax.numpy as jnp
from jax import lax
from jax.experimental import pallas as pl
from jax.experimental.pallas import tpu as pltpu
from jax.experimental.pallas import tpu_sc as plsc

NC = 2
NS = 16
NW = NC * NS
CK = 32
NRB = 8
NB = 12
LANES = 16


ZROWS = CK


def _make_sc_aggregate(n_pad: int, n_chunks: int, feat: int):
  rows_per_tile = n_pad // NS
  mesh = plsc.VectorSubcoreMesh(
      core_axis_name="c", subcore_axis_name="s", num_cores=NC,
      num_subcores=NS)

  @functools.partial(
      pl.kernel,
      out_type=jax.ShapeDtypeStruct((NC * n_pad, feat), jnp.float32),
      mesh=mesh,
      scratch_types=[
          pltpu.VMEM((NB, 2, CK), jnp.int32),
          pltpu.VMEM((NB, CK), jnp.float32),
          pltpu.VMEM((NRB, CK, feat), jnp.float32),
          pltpu.VMEM_SHARED((n_pad, feat), jnp.float32),
          pltpu.SemaphoreType.DMA,
          pltpu.SemaphoreType.DMA,
          pltpu.SemaphoreType.DMA,
      ],
  )
  def sc_aggregate(h_hbm, eidx_hbm, ew_hbm, out_hbm, ebuf, wbuf, rows_v,
                   agg_sh, esem, gsem, ssem):
    cid = lax.axis_index("c")
    sid = lax.axis_index("s")
    wid = sid * NC + cid

    def fetch_idx(c):
      pltpu.async_copy(eidx_hbm.at[wid, c], ebuf.at[lax.rem(c, NB)], esem)
      pltpu.async_copy(ew_hbm.at[wid, c], wbuf.at[lax.rem(c, NB)], esem)

    def wait_idx(c):
      pltpu.make_async_copy(eidx_hbm.at[wid, c],
                            ebuf.at[lax.rem(c, NB)], esem).wait()
      pltpu.make_async_copy(ew_hbm.at[wid, c],
                            wbuf.at[lax.rem(c, NB)], esem).wait()

    def start_gather(c, b):
      pltpu.async_copy(h_hbm.at[ebuf.at[lax.rem(c, NB), 0]],
                       rows_v.at[b], gsem)

    def wait_gather(c, b):
      pltpu.make_async_copy(h_hbm.at[ebuf.at[lax.rem(c, NB), 0]],
                            rows_v.at[b], gsem).wait()

    def start_scatter(c, b):
      pltpu.async_copy(rows_v.at[b], agg_sh.at[ebuf.at[lax.rem(c, NB), 1]],
                       ssem, add=True)

    def wait_scatter(c, b):
      pltpu.make_async_copy(rows_v.at[b],
                            agg_sh.at[ebuf.at[lax.rem(c, NB), 1]],
                            ssem).wait()

    zvec = jnp.zeros((LANES,), jnp.float32)

    def zfill(i, carry):
      for k in range(feat // LANES):
        rows_v[0, i, pl.ds(k * LANES, LANES)] = zvec
      return carry

    lax.fori_loop(0, ZROWS, zfill, 0)
    for i in range(rows_per_tile // ZROWS):
      pltpu.sync_copy(rows_v.at[0], agg_sh.at[pl.ds(sid * rows_per_tile
                                                    + i * ZROWS, ZROWS)])
    plsc.subcore_barrier()

    for c in range(min(7, n_chunks)):
      fetch_idx(c)
    for c in range(min(5, n_chunks)):
      wait_idx(c)
      start_gather(c, c)

    def chunk_body(c, carry):
      b = lax.rem(c, NRB)
      wait_gather(c, b)

      @pl.when(c >= 2)
      def _():
        wait_scatter(c - 2, lax.rem(c - 2, NRB))

      @pl.when(c + 5 < n_chunks)
      def _():
        wait_idx(c + 5)
        start_gather(c + 5, lax.rem(c + 5, NRB))

      @pl.when(c + 7 < n_chunks)
      def _():
        fetch_idx(c + 7)

      wslot = lax.rem(c, NB)

      @plsc.parallel_loop(0, CK)
      def scale_edge(e):
        g = lax.div(e, LANES) * LANES
        w16 = wbuf[wslot, pl.ds(g, LANES)]
        wj = lax.gather(
            w16, jnp.full((LANES, 1), e - g, jnp.int32),
            lax.GatherDimensionNumbers(offset_dims=(),
                                       collapsed_slice_dims=(0,),
                                       start_index_map=(0,)),
            slice_sizes=(1,),
            mode=lax.GatherScatterMode.PROMISE_IN_BOUNDS)
        vals = [rows_v[b, e, pl.ds(k * LANES, LANES)]
                for k in range(feat // LANES)]
        prods = [v * wj for v in vals]
        for k in range(feat // LANES):
          rows_v[b, e, pl.ds(k * LANES, LANES)] = prods[k]

      start_scatter(c, b)
      return carry

    lax.fori_loop(0, n_chunks, chunk_body, 0)
    for c in range(max(0, n_chunks - 2), n_chunks):
      wait_scatter(c, c % NRB)
    plsc.subcore_barrier()

    pltpu.sync_copy(
        agg_sh.at[pl.ds(sid * rows_per_tile, rows_per_tile)],
        out_hbm.at[pl.ds(cid * n_pad + sid * rows_per_tile,
                         rows_per_tile)])

  return sc_aggregate


_BLK = 512


def _layer_body(h_ref, a0_ref, a1_ref, w_ref, b_ref, o_ref):
  x = h_ref[...] + a0_ref[...] + a1_ref[...]
  y = lax.dot_general(x, w_ref[...], (((1,), (1,)), ((), ())),
                      preferred_element_type=jnp.float32)
  o_ref[...] = jnp.maximum(y + b_ref[...], 0.0)


def _tc_layer(h, agg, w, b):
  n_pad, feat = h.shape
  grid = n_pad // _BLK
  off = n_pad // _BLK
  return pl.pallas_call(
      _layer_body,
      grid=(grid,),
      in_specs=[
          pl.BlockSpec((_BLK, feat), lambda i: (i, 0)),
          pl.BlockSpec((_BLK, feat), lambda i: (i, 0)),
          pl.BlockSpec((_BLK, feat), lambda i: (i + off, 0)),
          pl.BlockSpec(w.shape, lambda i: (0, 0)),
          pl.BlockSpec((1, feat), lambda i: (0, 0)),
      ],
      out_specs=pl.BlockSpec((_BLK, feat), lambda i: (i, 0)),
      out_shape=jax.ShapeDtypeStruct((n_pad, feat), jnp.float32),
  )(h, agg, agg, w, b.reshape(1, feat))


def _readout_body(h1_ref, h2_ref, h3_ref, wr_ref, br_ref, o_ref):
  feat = h1_ref.shape[1]
  dn = (((1,), (1,)), ((), ()))
  y = lax.dot_general(jnp.maximum(h1_ref[...], 0.0), wr_ref[:, 0:feat],
                      dn, preferred_element_type=jnp.float32)
  y += lax.dot_general(jnp.maximum(h2_ref[...], 0.0),
                       wr_ref[:, feat:2 * feat], dn,
                       preferred_element_type=jnp.float32)
  y += lax.dot_general(jnp.maximum(h3_ref[...], 0.0),
                       wr_ref[:, 2 * feat:3 * feat], dn,
                       preferred_element_type=jnp.float32)
  o_ref[...] = y + br_ref[...]


def _tc_readout(h1, h2, h3, wr, br):
  n, feat = h1.shape
  grid = n // _BLK
  return pl.pallas_call(
      _readout_body,
      grid=(grid,),
      in_specs=[
          pl.BlockSpec((_BLK, feat), lambda i: (i, 0)),
          pl.BlockSpec((_BLK, feat), lambda i: (i, 0)),
          pl.BlockSpec((_BLK, feat), lambda i: (i, 0)),
          pl.BlockSpec(wr.shape, lambda i: (0, 0)),
          pl.BlockSpec((1, feat), lambda i: (0, 0)),
      ],
      out_specs=pl.BlockSpec((_BLK, feat), lambda i: (i, 0)),
      out_shape=jax.ShapeDtypeStruct((n, feat), jnp.float32),
  )(h1, h2, h3, wr, br.reshape(1, feat))


def kernel(node_embed, edge_index, edge_weight, W0, b0, W1, b1, W2, b2,
           Wr, br):
  n, feat = node_embed.shape
  e = edge_index.shape[1]
  n_chunks = -(-e // (NW * CK))
  e_pad = NW * n_chunks * CK
  n_pad = -(-n // (NS * ZROWS)) * NS * ZROWS

  src = jnp.pad(edge_index[0], (0, e_pad - e)).reshape(NW, n_chunks, CK)
  dst = jnp.pad(edge_index[1], (0, e_pad - e)).reshape(NW, n_chunks, CK)
  eidx = jnp.stack([src, dst], axis=2)
  ew = jnp.pad(edge_weight, (0, e_pad - e)).reshape(NW, n_chunks, CK)

  sc_aggregate = _make_sc_aggregate(n_pad, n_chunks, feat)

  def gin_layer(h, wmat, bvec):
    agg = sc_aggregate(h, eidx, ew)
    return _tc_layer(h, agg, wmat, bvec)

  h0 = jnp.pad(node_embed, ((0, n_pad - n), (0, 0)))
  h1 = gin_layer(h0, W0, b0)
  h2 = gin_layer(h1, W1, b1)
  h3 = gin_layer(h2, W2, b2)
  return _tc_readout(h1, h2, h3, Wr, br)[:n]


# speedup vs baseline: 1.0049x; 1.0045x over previous
import functools

import jax
import jax.numpy as jnp
from jax import lax
from jax.experimental import pallas as pl
from jax.experimental.pallas import tpu as pltpu
from jax.experimental.pallas import tpu_sc as plsc

NC = 2
NS = 16
NW = NC * NS
CK = 32
NRB = 8
NB = 12
LANES = 16


ZROWS = CK


def _make_sc_aggregate(n_pad: int, n_chunks: int, feat: int):
  rows_per_tile = n_pad // NS
  mesh = plsc.VectorSubcoreMesh(
      core_axis_name="c", subcore_axis_name="s", num_cores=NC,
      num_subcores=NS)

  @functools.partial(
      pl.kernel,
      out_type=jax.ShapeDtypeStruct((NC * n_pad, feat), jnp.float32),
      mesh=mesh,
      scratch_types=[
          pltpu.VMEM((NB, 2, CK), jnp.int32),
          pltpu.VMEM((NB, CK), jnp.float32),
          pltpu.VMEM((NRB, CK, feat), jnp.float32),
          pltpu.VMEM_SHARED((n_pad, feat), jnp.float32),
          pltpu.SemaphoreType.DMA,
          pltpu.SemaphoreType.DMA,
          pltpu.SemaphoreType.DMA,
      ],
  )
  def sc_aggregate(h_hbm, eidx_hbm, ew_hbm, out_hbm, ebuf, wbuf, rows_v,
                   agg_sh, esem, gsem, ssem):
    cid = lax.axis_index("c")
    sid = lax.axis_index("s")
    wid = sid * NC + cid

    def fetch_idx(c):
      pltpu.async_copy(eidx_hbm.at[wid, c], ebuf.at[lax.rem(c, NB)], esem)
      pltpu.async_copy(ew_hbm.at[wid, c], wbuf.at[lax.rem(c, NB)], esem)

    def wait_idx(c):
      pltpu.make_async_copy(eidx_hbm.at[wid, c],
                            ebuf.at[lax.rem(c, NB)], esem).wait()
      pltpu.make_async_copy(ew_hbm.at[wid, c],
                            wbuf.at[lax.rem(c, NB)], esem).wait()

    def start_gather(c, b):
      pltpu.async_copy(h_hbm.at[ebuf.at[lax.rem(c, NB), 0]],
                       rows_v.at[b], gsem)

    def wait_gather(c, b):
      pltpu.make_async_copy(h_hbm.at[ebuf.at[lax.rem(c, NB), 0]],
                            rows_v.at[b], gsem).wait()

    def start_scatter(c, b):
      pltpu.async_copy(rows_v.at[b], agg_sh.at[ebuf.at[lax.rem(c, NB), 1]],
                       ssem, add=True)

    def wait_scatter(c, b):
      pltpu.make_async_copy(rows_v.at[b],
                            agg_sh.at[ebuf.at[lax.rem(c, NB), 1]],
                            ssem).wait()

    for c in range(min(7, n_chunks)):
      fetch_idx(c)
    for c in range(min(5, n_chunks)):
      wait_idx(c)
      start_gather(c, c)

    zvec = jnp.zeros((LANES,), jnp.float32)

    def zfill(i, carry):
      for k in range(feat // LANES):
        rows_v[NRB - 1, i, pl.ds(k * LANES, LANES)] = zvec
      return carry

    lax.fori_loop(0, ZROWS, zfill, 0)
    for i in range(rows_per_tile // ZROWS):
      pltpu.sync_copy(rows_v.at[NRB - 1],
                      agg_sh.at[pl.ds(sid * rows_per_tile
                                      + i * ZROWS, ZROWS)])
    plsc.subcore_barrier()

    def chunk_body(c, carry):
      b = lax.rem(c, NRB)
      wait_gather(c, b)

      @pl.when(c >= 2)
      def _():
        wait_scatter(c - 2, lax.rem(c - 2, NRB))

      @pl.when(c + 5 < n_chunks)
      def _():
        wait_idx(c + 5)
        start_gather(c + 5, lax.rem(c + 5, NRB))

      @pl.when(c + 7 < n_chunks)
      def _():
        fetch_idx(c + 7)

      wslot = lax.rem(c, NB)

      @plsc.parallel_loop(0, CK)
      def scale_edge(e):
        g = lax.div(e, LANES) * LANES
        w16 = wbuf[wslot, pl.ds(g, LANES)]
        wj = lax.gather(
            w16, jnp.full((LANES, 1), e - g, jnp.int32),
            lax.GatherDimensionNumbers(offset_dims=(),
                                       collapsed_slice_dims=(0,),
                                       start_index_map=(0,)),
            slice_sizes=(1,),
            mode=lax.GatherScatterMode.PROMISE_IN_BOUNDS)
        vals = [rows_v[b, e, pl.ds(k * LANES, LANES)]
                for k in range(feat // LANES)]
        prods = [v * wj for v in vals]
        for k in range(feat // LANES):
          rows_v[b, e, pl.ds(k * LANES, LANES)] = prods[k]

      start_scatter(c, b)
      return carry

    lax.fori_loop(0, n_chunks, chunk_body, 0)
    for c in range(max(0, n_chunks - 2), n_chunks):
      wait_scatter(c, c % NRB)
    plsc.subcore_barrier()

    pltpu.sync_copy(
        agg_sh.at[pl.ds(sid * rows_per_tile, rows_per_tile)],
        out_hbm.at[pl.ds(cid * n_pad + sid * rows_per_tile,
                         rows_per_tile)])

  return sc_aggregate


_BLK = 512


def _layer_body(h_ref, a0_ref, a1_ref, w_ref, b_ref, o_ref):
  x = h_ref[...] + a0_ref[...] + a1_ref[...]
  y = lax.dot_general(x, w_ref[...], (((1,), (1,)), ((), ())),
                      preferred_element_type=jnp.float32)
  o_ref[...] = jnp.maximum(y + b_ref[...], 0.0)


def _tc_layer(h, agg, w, b):
  n_pad, feat = h.shape
  grid = n_pad // _BLK
  off = n_pad // _BLK
  return pl.pallas_call(
      _layer_body,
      grid=(grid,),
      in_specs=[
          pl.BlockSpec((_BLK, feat), lambda i: (i, 0)),
          pl.BlockSpec((_BLK, feat), lambda i: (i, 0)),
          pl.BlockSpec((_BLK, feat), lambda i: (i + off, 0)),
          pl.BlockSpec(w.shape, lambda i: (0, 0)),
          pl.BlockSpec((1, feat), lambda i: (0, 0)),
      ],
      out_specs=pl.BlockSpec((_BLK, feat), lambda i: (i, 0)),
      out_shape=jax.ShapeDtypeStruct((n_pad, feat), jnp.float32),
  )(h, agg, agg, w, b.reshape(1, feat))


def _readout_body(h1_ref, h2_ref, h3_ref, wr_ref, br_ref, o_ref):
  feat = h1_ref.shape[1]
  dn = (((1,), (1,)), ((), ()))
  y = lax.dot_general(jnp.maximum(h1_ref[...], 0.0), wr_ref[:, 0:feat],
                      dn, preferred_element_type=jnp.float32)
  y += lax.dot_general(jnp.maximum(h2_ref[...], 0.0),
                       wr_ref[:, feat:2 * feat], dn,
                       preferred_element_type=jnp.float32)
  y += lax.dot_general(jnp.maximum(h3_ref[...], 0.0),
                       wr_ref[:, 2 * feat:3 * feat], dn,
                       preferred_element_type=jnp.float32)
  o_ref[...] = y + br_ref[...]


def _tc_readout(h1, h2, h3, wr, br):
  n, feat = h1.shape
  grid = n // _BLK
  return pl.pallas_call(
      _readout_body,
      grid=(grid,),
      in_specs=[
          pl.BlockSpec((_BLK, feat), lambda i: (i, 0)),
          pl.BlockSpec((_BLK, feat), lambda i: (i, 0)),
          pl.BlockSpec((_BLK, feat), lambda i: (i, 0)),
          pl.BlockSpec(wr.shape, lambda i: (0, 0)),
          pl.BlockSpec((1, feat), lambda i: (0, 0)),
      ],
      out_specs=pl.BlockSpec((_BLK, feat), lambda i: (i, 0)),
      out_shape=jax.ShapeDtypeStruct((n, feat), jnp.float32),
  )(h1, h2, h3, wr, br.reshape(1, feat))


def kernel(node_embed, edge_index, edge_weight, W0, b0, W1, b1, W2, b2,
           Wr, br):
  n, feat = node_embed.shape
  e = edge_index.shape[1]
  n_chunks = -(-e // (NW * CK))
  e_pad = NW * n_chunks * CK
  n_pad = -(-n // (NS * ZROWS)) * NS * ZROWS

  src = jnp.pad(edge_index[0], (0, e_pad - e)).reshape(NW, n_chunks, CK)
  dst = jnp.pad(edge_index[1], (0, e_pad - e)).reshape(NW, n_chunks, CK)
  eidx = jnp.stack([src, dst], axis=2)
  ew = jnp.pad(edge_weight, (0, e_pad - e)).reshape(NW, n_chunks, CK)

  sc_aggregate = _make_sc_aggregate(n_pad, n_chunks, feat)

  def gin_layer(h, wmat, bvec):
    agg = sc_aggregate(h, eidx, ew)
    return _tc_layer(h, agg, wmat, bvec)

  h0 = jnp.pad(node_embed, ((0, n_pad - n), (0, 0)))
  h1 = gin_layer(h0, W0, b0)
  h2 = gin_layer(h1, W1, b1)
  h3 = gin_layer(h2, W2, b2)
  return _tc_readout(h1, h2, h3, Wr, br)[:n]
